# Initial kernel scaffold; baseline (speedup 1.0000x reference)
import jax
import jax.numpy as jnp
from jax.experimental import pallas as pl


def kernel(x, W1, W2, W3, W4, W5, We1, We2, We3, Wf0, bf0, Wf1, bf1, Wf2, bf2, Wf3, bf3, Wsem, bsem):
    raise NotImplementedError("write your pallas kernel here")


# speedup vs baseline: 9.6314x; 9.6314x over previous
import functools

import jax
import jax.numpy as jnp
from jax import lax
from jax.experimental import pallas as pl
from jax.experimental.pallas import tpu as pltpu
from jax.experimental.pallas import tpu_sc as plsc

_B, _N, _K = 8, 2048, 20
_BN = _B * _N
_W = 128
_R = 256
_RE = 128
_RT = 512
_NW = 32
_CH = 128


def _knn_body(c, xf_ref, xr_ref, gidx_ref):
    b = pl.program_id(0)
    xf = xf_ref[0][:, 0:c]
    xr = xr_ref[0][:, 0:c]
    g = lax.dot_general(xr.astype(jnp.bfloat16), xf.astype(jnp.bfloat16),
                        (((1,), (1,)), ((), ())),
                        preferred_element_type=jnp.float32)
    inner = -2.0 * g
    ones = jnp.ones((1, c), jnp.float32)
    xx = lax.dot_general(ones, xf * xf, (((1,), (1,)), ((), ())),
                         preferred_element_type=jnp.float32,
                         precision=lax.Precision.HIGHEST)
    xxr = lax.dot_general(xr * xr, ones, (((1,), (1,)), ((), ())),
                          preferred_element_type=jnp.float32,
                          precision=lax.Precision.HIGHEST)
    s = (-xxr) - inner
    s = s - xx
    col = lax.broadcasted_iota(jnp.int32, s.shape, 1)
    picks = []
    for _ in range(_K):
        m = jnp.max(s, axis=1, keepdims=True)
        cand = jnp.where(s == m, col, _N)
        amin = jnp.min(cand, axis=1, keepdims=True)
        picks.append(amin)
        s = jnp.where(col == amin, -jnp.inf, s)
    gidx_ref[0] = jnp.concatenate(picks, axis=1) + b * _N


def _knn(xpad, c):
    x3 = xpad.reshape(_B, _N, _W)
    return pl.pallas_call(
        functools.partial(_knn_body, c),
        grid=(_B, _N // _R),
        in_specs=[
            pl.BlockSpec((1, _N, _W), lambda b, r: (b, 0, 0)),
            pl.BlockSpec((1, _R, _W), lambda b, r: (b, r, 0)),
        ],
        out_specs=pl.BlockSpec((1, _R, _K), lambda b, r: (b, r, 0)),
        out_shape=jax.ShapeDtypeStruct((_B, _N, _K), jnp.int32),
    )(x3, x3)


def _make_sc_gather():
    edges = _BN * _K
    per_w = edges // _NW
    nch = per_w // _CH
    mesh = plsc.VectorSubcoreMesh(core_axis_name="c", subcore_axis_name="s")

    @functools.partial(
        pl.kernel,
        mesh=mesh,
        out_type=jax.ShapeDtypeStruct((edges, _W), jnp.float32),
        scratch_types=[
            pltpu.VMEM((_CH,), jnp.int32),
            pltpu.VMEM((_CH, _W), jnp.float32),
            pltpu.SemaphoreType.DMA,
        ],
    )
    def gather(x_hbm, idx_hbm, g_hbm, idx_v, rows_v, sem):
        wid = lax.axis_index("s") * 2 + lax.axis_index("c")
        base = wid * per_w

        def chunk(t, carry):
            e0 = base + t * _CH
            pltpu.sync_copy(idx_hbm.at[pl.ds(e0, _CH)], idx_v)
            pltpu.async_copy(x_hbm.at[idx_v], rows_v, sem).wait()
            pltpu.sync_copy(rows_v, g_hbm.at[pl.ds(e0, _CH)])
            return carry

        lax.fori_loop(0, nch, chunk, 0)

    return gather


_sc_gather = None


def _gather_rows(xpad, idx_jmajor):
    global _sc_gather
    if _sc_gather is None:
        _sc_gather = _make_sc_gather()
    return _sc_gather(xpad, idx_jmajor)


def _edge_body(c, oc, ocp, g_ref, xr_ref, w_ref, out_ref):
    g3 = g_ref[:, :, 0:c]
    xr = xr_ref[:, 0:c]
    diff = g3 - xr[None]
    cb = jnp.broadcast_to(xr[None], g3.shape)
    f = jnp.concatenate([diff, cb], axis=2).astype(jnp.bfloat16)
    f2 = f.reshape(_K * _RE, 2 * c)
    y = lax.dot_general(f2, w_ref[...].astype(jnp.bfloat16),
                        (((1,), (1,)), ((), ())),
                        preferred_element_type=jnp.float32)
    m = jnp.max(y.reshape(_K, _RE, oc), axis=0)
    m = m / jnp.sqrt(1.0 + 1e-5)
    m = jnp.where(m >= 0.0, m, 0.2 * m)
    if ocp > oc:
        out_ref[:, 0:oc] = m
        out_ref[:, oc:ocp] = jnp.zeros((_RE, ocp - oc), jnp.float32)
    else:
        out_ref[...] = m


def _edge_conv(g, xpad, w):
    oc, c2 = w.shape
    c = c2 // 2
    ocp = max(oc, _W)
    g3 = g.reshape(_K, _BN, _W)
    return pl.pallas_call(
        functools.partial(_edge_body, c, oc, ocp),
        grid=(_BN // _RE,),
        in_specs=[
            pl.BlockSpec((_K, _RE, _W), lambda t: (0, t, 0)),
            pl.BlockSpec((_RE, _W), lambda t: (t, 0)),
            pl.BlockSpec((oc, 2 * c), lambda t: (0, 0)),
        ],
        out_specs=pl.BlockSpec((_RE, ocp), lambda t: (t, 0)),
        out_shape=jax.ShapeDtypeStruct((_BN, ocp), jnp.float32),
    )(g3, xpad, w)


def _tail_body(x1_ref, x2_ref, x3_ref, x4_ref, w5_ref, we1_ref, we2_ref,
               we3_ref, wf0_ref, bf0_ref, wf1_ref, bf1_ref, wf2_ref, bf2_ref,
               wf3_ref, bf3_ref, wsem_ref, bsem_ref, out_ref):
    xc = jnp.concatenate(
        [x1_ref[:, 0:64], x2_ref[:, 0:64], x3_ref[...], x4_ref[...]], axis=1)

    def mm_t(w, xpm):
        return lax.dot_general(w, xpm, (((1,), (1,)), ((), ())),
                               preferred_element_type=jnp.float32,
                               precision=lax.Precision.HIGHEST)

    def mm(w, xcm):
        return lax.dot_general(w, xcm, (((1,), (0,)), ((), ())),
                               preferred_element_type=jnp.float32,
                               precision=lax.Precision.HIGHEST)

    def lr(y):
        return jnp.where(y >= 0.0, y, 0.2 * y)

    h = lr(mm_t(w5_ref[...], xc))
    e = lr(mm(we1_ref[...], h))
    e = lr(mm(we2_ref[...], e))
    e = lr(mm(we3_ref[...], e))
    f0 = lr(mm(wf0_ref[...], h) + bf0_ref[...])
    f1 = lr(mm(wf1_ref[...], h) + bf1_ref[...])
    f2 = lr(mm(wf2_ref[...], h) + bf2_ref[...])
    f3 = lr(mm(wf3_ref[...], h) + bf3_ref[...])
    sem = mm(wsem_ref[...], f3) + bsem_ref[...]
    out_ref[0, 0:128, :] = e
    out_ref[0, 128:192, :] = f0
    out_ref[0, 192:256, :] = f1
    out_ref[0, 256:320, :] = f2
    out_ref[0, 320:448, :] = f3
    out_ref[0, 448:468, :] = sem


def _tail(x1, x2, x3, x4, w5, we1, we2, we3, wf0, bf0, wf1, bf1, wf2, bf2,
          wf3, bf3, wsem, bsem):
    tiles = _N // _RT

    def row_spec(ch):
        return pl.BlockSpec((_RT, ch), lambda b, t: (b * tiles + t, 0))

    def full(a):
        return pl.BlockSpec(a.shape, lambda b, t: tuple(0 for _ in a.shape))

    return pl.pallas_call(
        _tail_body,
        grid=(_B, tiles),
        in_specs=[row_spec(128), row_spec(128), row_spec(128), row_spec(256),
                  full(w5), full(we1), full(we2), full(we3),
                  full(wf0), full(bf0), full(wf1), full(bf1),
                  full(wf2), full(bf2), full(wf3), full(bf3),
                  full(wsem), full(bsem)],
        out_specs=pl.BlockSpec((1, 468, _RT), lambda b, t: (b, 0, t)),
        out_shape=jax.ShapeDtypeStruct((_B, 468, _N), jnp.float32),
    )(x1, x2, x3, x4, w5, we1, we2, we3, wf0, bf0, wf1, bf1, wf2, bf2,
      wf3, bf3, wsem, bsem)


_S = float((1.0 + 1e-5) ** -0.5)


def kernel(x, W1, W2, W3, W4, W5, We1, We2, We3, Wf0, bf0, Wf1, bf1, Wf2, bf2,
           Wf3, bf3, Wsem, bsem):
    c0 = x.shape[1]
    xpad = jnp.pad(jnp.transpose(x, (0, 2, 1)).reshape(_BN, c0),
                   ((0, 0), (0, _W - c0)))
    feats = []
    for w in (W1, W2, W3, W4):
        c = w.shape[1] // 2
        gidx = _knn(xpad, c)
        jm = jnp.transpose(gidx.reshape(_BN, _K)).reshape(_BN * _K)
        g = _gather_rows(xpad, jm)
        xpad = _edge_conv(g, xpad, w)
        feats.append(xpad)
    x1, x2, x3, x4 = feats
    return _tail(
        x1, x2, x3, x4,
        W5 * _S, We1 * _S, We2 * _S, We3 * _S,
        Wf0 * _S, (bf0 * _S)[:, None], Wf1 * _S, (bf1 * _S)[:, None],
        Wf2 * _S, (bf2 * _S)[:, None], Wf3 * _S, (bf3 * _S)[:, None],
        Wsem, bsem[:, None])


# speedup vs baseline: 12.6741x; 1.3159x over previous
import functools

import jax
import jax.numpy as jnp
from jax import lax
from jax.experimental import pallas as pl
from jax.experimental.pallas import tpu as pltpu
from jax.experimental.pallas import tpu_sc as plsc

_B, _N, _K = 8, 2048, 20
_BN = _B * _N
_W = 128
_R = 256
_RE = 128
_RT = 512
_NW = 32
_CH = 128


def _knn_body(c, xf_ref, xr_ref, gidx_ref):
    b = pl.program_id(0)
    xf = xf_ref[0][:, 0:c]
    xr = xr_ref[0][:, 0:c]
    g = lax.dot_general(xr.astype(jnp.bfloat16), xf.astype(jnp.bfloat16),
                        (((1,), (1,)), ((), ())),
                        preferred_element_type=jnp.float32)
    inner = -2.0 * g
    ones = jnp.ones((1, c), jnp.float32)
    xx = lax.dot_general(ones, xf * xf, (((1,), (1,)), ((), ())),
                         preferred_element_type=jnp.float32,
                         precision=lax.Precision.HIGHEST)
    xxr = lax.dot_general(xr * xr, ones, (((1,), (1,)), ((), ())),
                          preferred_element_type=jnp.float32,
                          precision=lax.Precision.HIGHEST)
    s = (-xxr) - inner
    s = s - xx
    colf = lax.broadcasted_iota(jnp.int32, s.shape, 1).astype(jnp.float32)
    picks = []
    for _ in range(_K):
        m = jnp.max(s, axis=1, keepdims=True)
        cand = jnp.where(s == m, colf, 4096.0)
        amin = jnp.min(cand, axis=1, keepdims=True)
        picks.append(amin)
        s = jnp.where(colf == amin, -jnp.inf, s)
    gidx_ref[0] = (jnp.concatenate(picks, axis=1).astype(jnp.int32)
                   + b * _N)


def _knn(xpad, c):
    x3 = xpad.reshape(_B, _N, _W)
    return pl.pallas_call(
        functools.partial(_knn_body, c),
        grid=(_B, _N // _R),
        in_specs=[
            pl.BlockSpec((1, _N, _W), lambda b, r: (b, 0, 0)),
            pl.BlockSpec((1, _R, _W), lambda b, r: (b, r, 0)),
        ],
        out_specs=pl.BlockSpec((1, _R, _K), lambda b, r: (b, r, 0)),
        out_shape=jax.ShapeDtypeStruct((_B, _N, _K), jnp.int32),
    )(x3, x3)


def _make_sc_gather():
    edges = _BN * _K
    per_w = edges // _NW
    nch = per_w // _CH
    nbuf = 4
    mesh = plsc.VectorSubcoreMesh(core_axis_name="c", subcore_axis_name="s")

    @functools.partial(
        pl.kernel,
        mesh=mesh,
        out_type=jax.ShapeDtypeStruct((edges, _W), jnp.float32),
        scratch_types=(
            [pltpu.VMEM((per_w,), jnp.int32),
             pltpu.VMEM((nbuf, _CH, _W), jnp.float32)]
            + [pltpu.SemaphoreType.DMA] * (2 * nbuf)
        ),
    )
    def gather(x_hbm, idx_hbm, g_hbm, idx_all, rows, *sems):
        gsem, osem = sems[:nbuf], sems[nbuf:]
        wid = lax.axis_index("s") * 2 + lax.axis_index("c")
        base = wid * per_w
        pltpu.sync_copy(idx_hbm.at[pl.ds(base, per_w)], idx_all)
        gcp = [None] * nch
        ocp = [None] * nch

        def gfire(t):
            b = t % nbuf
            gcp[t] = pltpu.async_copy(
                x_hbm.at[idx_all.at[pl.ds(t * _CH, _CH)]], rows.at[b],
                gsem[b])

        gfire(0)
        gfire(1)
        for t in range(nch):
            b = t % nbuf
            gcp[t].wait()
            ocp[t] = pltpu.async_copy(
                rows.at[b], g_hbm.at[pl.ds(base + t * _CH, _CH)], osem[b])
            nxt = t + 2
            if nxt < nch:
                if nxt - nbuf >= 0:
                    ocp[nxt - nbuf].wait()
                gfire(nxt)
        for t in range(max(0, nch - nbuf), nch):
            ocp[t].wait()

    return gather


_sc_gather = None


def _gather_rows(xpad, idx_jmajor):
    global _sc_gather
    if _sc_gather is None:
        _sc_gather = _make_sc_gather()
    return _sc_gather(xpad, idx_jmajor)


def _edge_body(c, oc, ocp, g_ref, xr_ref, w_ref, out_ref):
    g3 = g_ref[:, :, 0:c]
    xr = xr_ref[:, 0:c]
    diff = g3 - xr[None]
    cb = jnp.broadcast_to(xr[None], g3.shape)
    f = jnp.concatenate([diff, cb], axis=2).astype(jnp.bfloat16)
    f2 = f.reshape(_K * _RE, 2 * c)
    y = lax.dot_general(f2, w_ref[...].astype(jnp.bfloat16),
                        (((1,), (1,)), ((), ())),
                        preferred_element_type=jnp.float32)
    m = jnp.max(y.reshape(_K, _RE, oc), axis=0)
    m = m / jnp.sqrt(1.0 + 1e-5)
    m = jnp.where(m >= 0.0, m, 0.2 * m)
    if ocp > oc:
        out_ref[:, 0:oc] = m
        out_ref[:, oc:ocp] = jnp.zeros((_RE, ocp - oc), jnp.float32)
    else:
        out_ref[...] = m


def _edge_conv(g, xpad, w):
    oc, c2 = w.shape
    c = c2 // 2
    ocp = max(oc, _W)
    g3 = g.reshape(_K, _BN, _W)
    return pl.pallas_call(
        functools.partial(_edge_body, c, oc, ocp),
        grid=(_BN // _RE,),
        in_specs=[
            pl.BlockSpec((_K, _RE, _W), lambda t: (0, t, 0)),
            pl.BlockSpec((_RE, _W), lambda t: (t, 0)),
            pl.BlockSpec((oc, 2 * c), lambda t: (0, 0)),
        ],
        out_specs=pl.BlockSpec((_RE, ocp), lambda t: (t, 0)),
        out_shape=jax.ShapeDtypeStruct((_BN, ocp), jnp.float32),
    )(g3, xpad, w)


def _tail_body(x1_ref, x2_ref, x3_ref, x4_ref, w5_ref, we1_ref, we2_ref,
               we3_ref, wf0_ref, bf0_ref, wf1_ref, bf1_ref, wf2_ref, bf2_ref,
               wf3_ref, bf3_ref, wsem_ref, bsem_ref, out_ref):
    xc = jnp.concatenate(
        [x1_ref[:, 0:64], x2_ref[:, 0:64], x3_ref[...], x4_ref[...]], axis=1)

    def mm_t(w, xpm):
        return lax.dot_general(w, xpm, (((1,), (1,)), ((), ())),
                               preferred_element_type=jnp.float32,
                               precision=lax.Precision.HIGHEST)

    def mm(w, xcm):
        return lax.dot_general(w, xcm, (((1,), (0,)), ((), ())),
                               preferred_element_type=jnp.float32,
                               precision=lax.Precision.HIGHEST)

    def lr(y):
        return jnp.where(y >= 0.0, y, 0.2 * y)

    h = lr(mm_t(w5_ref[...], xc))
    e = lr(mm(we1_ref[...], h))
    e = lr(mm(we2_ref[...], e))
    e = lr(mm(we3_ref[...], e))
    f0 = lr(mm(wf0_ref[...], h) + bf0_ref[...])
    f1 = lr(mm(wf1_ref[...], h) + bf1_ref[...])
    f2 = lr(mm(wf2_ref[...], h) + bf2_ref[...])
    f3 = lr(mm(wf3_ref[...], h) + bf3_ref[...])
    sem = mm(wsem_ref[...], f3) + bsem_ref[...]
    out_ref[0, 0:128, :] = e
    out_ref[0, 128:192, :] = f0
    out_ref[0, 192:256, :] = f1
    out_ref[0, 256:320, :] = f2
    out_ref[0, 320:448, :] = f3
    out_ref[0, 448:468, :] = sem


def _tail(x1, x2, x3, x4, w5, we1, we2, we3, wf0, bf0, wf1, bf1, wf2, bf2,
          wf3, bf3, wsem, bsem):
    tiles = _N // _RT

    def row_spec(ch):
        return pl.BlockSpec((_RT, ch), lambda b, t: (b * tiles + t, 0))

    def full(a):
        return pl.BlockSpec(a.shape, lambda b, t: tuple(0 for _ in a.shape))

    return pl.pallas_call(
        _tail_body,
        grid=(_B, tiles),
        in_specs=[row_spec(128), row_spec(128), row_spec(128), row_spec(256),
                  full(w5), full(we1), full(we2), full(we3),
                  full(wf0), full(bf0), full(wf1), full(bf1),
                  full(wf2), full(bf2), full(wf3), full(bf3),
                  full(wsem), full(bsem)],
        out_specs=pl.BlockSpec((1, 468, _RT), lambda b, t: (b, 0, t)),
        out_shape=jax.ShapeDtypeStruct((_B, 468, _N), jnp.float32),
    )(x1, x2, x3, x4, w5, we1, we2, we3, wf0, bf0, wf1, bf1, wf2, bf2,
      wf3, bf3, wsem, bsem)


_S = float((1.0 + 1e-5) ** -0.5)


def kernel(x, W1, W2, W3, W4, W5, We1, We2, We3, Wf0, bf0, Wf1, bf1, Wf2, bf2,
           Wf3, bf3, Wsem, bsem):
    c0 = x.shape[1]
    xpad = jnp.pad(jnp.transpose(x, (0, 2, 1)).reshape(_BN, c0),
                   ((0, 0), (0, _W - c0)))
    feats = []
    for w in (W1, W2, W3, W4):
        c = w.shape[1] // 2
        gidx = _knn(xpad, c)
        jm = jnp.transpose(gidx.reshape(_BN, _K)).reshape(_BN * _K)
        g = _gather_rows(xpad, jm)
        xpad = _edge_conv(g, xpad, w)
        feats.append(xpad)
    x1, x2, x3, x4 = feats
    return _tail(
        x1, x2, x3, x4,
        W5 * _S, We1 * _S, We2 * _S, We3 * _S,
        Wf0 * _S, (bf0 * _S)[:, None], Wf1 * _S, (bf1 * _S)[:, None],
        Wf2 * _S, (bf2 * _S)[:, None], Wf3 * _S, (bf3 * _S)[:, None],
        Wsem, bsem[:, None])


# speedup vs baseline: 12.6832x; 1.0007x over previous
import functools

import jax
import jax.numpy as jnp
from jax import lax
from jax.experimental import pallas as pl
from jax.experimental.pallas import tpu as pltpu
from jax.experimental.pallas import tpu_sc as plsc

_B, _N, _K = 8, 2048, 20
_BN = _B * _N
_W = 128
_R = 256
_RE = 128
_RT = 512
_NW = 32
_CH = 128


def _knn_body(c, xf_ref, xr_ref, gidx_ref):
    b = pl.program_id(0)
    xf = xf_ref[0][:, 0:c]
    xr = xr_ref[0][:, 0:c]
    g = lax.dot_general(xr.astype(jnp.bfloat16), xf.astype(jnp.bfloat16),
                        (((1,), (1,)), ((), ())),
                        preferred_element_type=jnp.float32)
    inner = -2.0 * g
    ones = jnp.ones((1, c), jnp.float32)
    xx = lax.dot_general(ones, xf * xf, (((1,), (1,)), ((), ())),
                         preferred_element_type=jnp.float32,
                         precision=lax.Precision.HIGHEST)
    xxr = lax.dot_general(xr * xr, ones, (((1,), (1,)), ((), ())),
                          preferred_element_type=jnp.float32,
                          precision=lax.Precision.HIGHEST)
    s = (-xxr) - inner
    s = s - xx
    colf = lax.broadcasted_iota(jnp.int32, s.shape, 1).astype(jnp.float32)
    picks = []
    for _ in range(_K):
        m = jnp.max(s, axis=1, keepdims=True)
        cand = jnp.where(s == m, colf, 4096.0)
        amin = jnp.min(cand, axis=1, keepdims=True)
        picks.append(amin)
        s = jnp.where(colf == amin, -jnp.inf, s)
    gidx_ref[0] = (jnp.concatenate(picks, axis=1).astype(jnp.int32)
                   + b * _N)


def _knn(xpad, c):
    x3 = xpad.reshape(_B, _N, _W)
    return pl.pallas_call(
        functools.partial(_knn_body, c),
        grid=(_B, _N // _R),
        in_specs=[
            pl.BlockSpec((1, _N, _W), lambda b, r: (b, 0, 0)),
            pl.BlockSpec((1, _R, _W), lambda b, r: (b, r, 0)),
        ],
        out_specs=pl.BlockSpec((1, _R, _K), lambda b, r: (b, r, 0)),
        out_shape=jax.ShapeDtypeStruct((_B, _N, _K), jnp.int32),
    )(x3, x3)


def _make_sc_gather():
    edges = _BN * _K
    per_w = edges // _NW
    nch = per_w // _CH
    nbuf = 4
    mesh = plsc.VectorSubcoreMesh(core_axis_name="c", subcore_axis_name="s")

    @functools.partial(
        pl.kernel,
        mesh=mesh,
        out_type=jax.ShapeDtypeStruct((edges, _W), jnp.float32),
        scratch_types=(
            [pltpu.VMEM((per_w,), jnp.int32),
             pltpu.VMEM((nbuf, _CH, _W), jnp.float32)]
            + [pltpu.SemaphoreType.DMA] * (2 * nbuf)
        ),
    )
    def gather(x_hbm, idx_hbm, g_hbm, idx_all, rows, *sems):
        gsem, osem = sems[:nbuf], sems[nbuf:]
        wid = lax.axis_index("s") * 2 + lax.axis_index("c")
        base = wid * per_w
        pltpu.sync_copy(idx_hbm.at[pl.ds(base, per_w)], idx_all)
        gcp = [None] * nch
        ocp = [None] * nch

        def gfire(t):
            b = t % nbuf
            gcp[t] = pltpu.async_copy(
                x_hbm.at[idx_all.at[pl.ds(t * _CH, _CH)]], rows.at[b],
                gsem[b])

        gfire(0)
        gfire(1)
        gfire(2)
        for t in range(nch):
            b = t % nbuf
            gcp[t].wait()
            ocp[t] = pltpu.async_copy(
                rows.at[b], g_hbm.at[pl.ds(base + t * _CH, _CH)], osem[b])
            nxt = t + 3
            if nxt < nch:
                if nxt - nbuf >= 0:
                    ocp[nxt - nbuf].wait()
                gfire(nxt)
        for t in range(max(0, nch - nbuf), nch):
            ocp[t].wait()

    return gather


_sc_gather = None


def _gather_rows(xpad, idx_jmajor):
    global _sc_gather
    if _sc_gather is None:
        _sc_gather = _make_sc_gather()
    return _sc_gather(xpad, idx_jmajor)


def _edge_body(c, oc, ocp, g_ref, xr_ref, w_ref, out_ref):
    g3 = g_ref[:, :, 0:c]
    xr = xr_ref[:, 0:c]
    diff = g3 - xr[None]
    cb = jnp.broadcast_to(xr[None], g3.shape)
    f = jnp.concatenate([diff, cb], axis=2).astype(jnp.bfloat16)
    f2 = f.reshape(_K * _RE, 2 * c)
    y = lax.dot_general(f2, w_ref[...].astype(jnp.bfloat16),
                        (((1,), (1,)), ((), ())),
                        preferred_element_type=jnp.float32)
    m = jnp.max(y.reshape(_K, _RE, oc), axis=0)
    m = m / jnp.sqrt(1.0 + 1e-5)
    m = jnp.where(m >= 0.0, m, 0.2 * m)
    if ocp > oc:
        out_ref[:, 0:oc] = m
        out_ref[:, oc:ocp] = jnp.zeros((_RE, ocp - oc), jnp.float32)
    else:
        out_ref[...] = m


def _edge_conv(g, xpad, w):
    oc, c2 = w.shape
    c = c2 // 2
    ocp = max(oc, _W)
    g3 = g.reshape(_K, _BN, _W)
    return pl.pallas_call(
        functools.partial(_edge_body, c, oc, ocp),
        grid=(_BN // _RE,),
        in_specs=[
            pl.BlockSpec((_K, _RE, _W), lambda t: (0, t, 0)),
            pl.BlockSpec((_RE, _W), lambda t: (t, 0)),
            pl.BlockSpec((oc, 2 * c), lambda t: (0, 0)),
        ],
        out_specs=pl.BlockSpec((_RE, ocp), lambda t: (t, 0)),
        out_shape=jax.ShapeDtypeStruct((_BN, ocp), jnp.float32),
    )(g3, xpad, w)


def _tail_body(x1_ref, x2_ref, x3_ref, x4_ref, w5_ref, we1_ref, we2_ref,
               we3_ref, wf0_ref, bf0_ref, wf1_ref, bf1_ref, wf2_ref, bf2_ref,
               wf3_ref, bf3_ref, wsem_ref, bsem_ref, out_ref):
    xc = jnp.concatenate(
        [x1_ref[:, 0:64], x2_ref[:, 0:64], x3_ref[...], x4_ref[...]], axis=1)

    def mm_t(w, xpm):
        return lax.dot_general(w, xpm, (((1,), (1,)), ((), ())),
                               preferred_element_type=jnp.float32,
                               precision=lax.Precision.HIGHEST)

    def mm(w, xcm):
        return lax.dot_general(w, xcm, (((1,), (0,)), ((), ())),
                               preferred_element_type=jnp.float32,
                               precision=lax.Precision.HIGHEST)

    def lr(y):
        return jnp.where(y >= 0.0, y, 0.2 * y)

    h = lr(mm_t(w5_ref[...], xc))
    e = lr(mm(we1_ref[...], h))
    e = lr(mm(we2_ref[...], e))
    e = lr(mm(we3_ref[...], e))
    f0 = lr(mm(wf0_ref[...], h) + bf0_ref[...])
    f1 = lr(mm(wf1_ref[...], h) + bf1_ref[...])
    f2 = lr(mm(wf2_ref[...], h) + bf2_ref[...])
    f3 = lr(mm(wf3_ref[...], h) + bf3_ref[...])
    sem = mm(wsem_ref[...], f3) + bsem_ref[...]
    out_ref[0, 0:128, :] = e
    out_ref[0, 128:192, :] = f0
    out_ref[0, 192:256, :] = f1
    out_ref[0, 256:320, :] = f2
    out_ref[0, 320:448, :] = f3
    out_ref[0, 448:468, :] = sem


def _tail(x1, x2, x3, x4, w5, we1, we2, we3, wf0, bf0, wf1, bf1, wf2, bf2,
          wf3, bf3, wsem, bsem):
    tiles = _N // _RT

    def row_spec(ch):
        return pl.BlockSpec((_RT, ch), lambda b, t: (b * tiles + t, 0))

    def full(a):
        return pl.BlockSpec(a.shape, lambda b, t: tuple(0 for _ in a.shape))

    return pl.pallas_call(
        _tail_body,
        grid=(_B, tiles),
        in_specs=[row_spec(128), row_spec(128), row_spec(128), row_spec(256),
                  full(w5), full(we1), full(we2), full(we3),
                  full(wf0), full(bf0), full(wf1), full(bf1),
                  full(wf2), full(bf2), full(wf3), full(bf3),
                  full(wsem), full(bsem)],
        out_specs=pl.BlockSpec((1, 468, _RT), lambda b, t: (b, 0, t)),
        out_shape=jax.ShapeDtypeStruct((_B, 468, _N), jnp.float32),
    )(x1, x2, x3, x4, w5, we1, we2, we3, wf0, bf0, wf1, bf1, wf2, bf2,
      wf3, bf3, wsem, bsem)


_S = float((1.0 + 1e-5) ** -0.5)


def kernel(x, W1, W2, W3, W4, W5, We1, We2, We3, Wf0, bf0, Wf1, bf1, Wf2, bf2,
           Wf3, bf3, Wsem, bsem):
    c0 = x.shape[1]
    xpad = jnp.pad(jnp.transpose(x, (0, 2, 1)).reshape(_BN, c0),
                   ((0, 0), (0, _W - c0)))
    feats = []
    for w in (W1, W2, W3, W4):
        c = w.shape[1] // 2
        gidx = _knn(xpad, c)
        jm = jnp.transpose(gidx.reshape(_BN, _K)).reshape(_BN * _K)
        g = _gather_rows(xpad, jm)
        xpad = _edge_conv(g, xpad, w)
        feats.append(xpad)
    x1, x2, x3, x4 = feats
    return _tail(
        x1, x2, x3, x4,
        W5 * _S, We1 * _S, We2 * _S, We3 * _S,
        Wf0 * _S, (bf0 * _S)[:, None], Wf1 * _S, (bf1 * _S)[:, None],
        Wf2 * _S, (bf2 * _S)[:, None], Wf3 * _S, (bf3 * _S)[:, None],
        Wsem, bsem[:, None])
